# Initial kernel scaffold; baseline (speedup 1.0000x reference)
import jax
import jax.numpy as jnp
from jax.experimental import pallas as pl


def kernel(node_feats, edge_feats, edge_index, Wp, bp, We1, be1, We2, be2, nn_bias, W_ih, W_hh, b_ih, b_hh):
    raise NotImplementedError("write your pallas kernel here")


# speedup vs baseline: 2.5196x; 2.5196x over previous
import functools

import jax
import jax.numpy as jnp
from jax import lax
from jax.experimental import pallas as pl
from jax.experimental.pallas import tpu as pltpu
from jax.experimental.pallas import tpu_sc as plsc

H = 32
EH = 64
NW = 32
CH = 128
GRP = 8


def _proj_body(nf_ref, wp_ref, bp_ref, out_ref):
    out_ref[...] = jnp.maximum(
        jnp.dot(nf_ref[...], wp_ref[...], preferred_element_type=jnp.float32)
        + bp_ref[...], 0.0)


def _msgs_body(ef_ref, hs_ref, we1_ref, be1_ref, we2p_ref, be2p_ref, r_ref, out_ref):
    a = jnp.maximum(
        jnp.dot(ef_ref[...], we1_ref[...], preferred_element_type=jnp.float32)
        + be1_ref[...], 0.0)
    c2 = jnp.dot(a, we2p_ref[...], preferred_element_type=jnp.float32) + be2p_ref[...]
    z = c2 * jnp.tile(hs_ref[...], (1, H))
    out_ref[...] = jnp.dot(z, r_ref[...], preferred_element_type=jnp.float32)


def _update_body(n_real, bn, agg0_ref, agg1_ref, hid_ref, nnb_ref,
                 wih_ref, bih_ref, whh_ref, bhh_ref, hnew_ref, pool_ref):
    i = pl.program_id(0)
    node = jnp.maximum(agg0_ref[...] + agg1_ref[...] + nnb_ref[...], 0.0)
    gi = jnp.dot(node, wih_ref[...], preferred_element_type=jnp.float32) + bih_ref[...]
    gh = jnp.dot(hid_ref[...], whh_ref[...], preferred_element_type=jnp.float32) + bhh_ref[...]
    r = jax.nn.sigmoid(gi[:, :H] + gh[:, :H])
    zg = jax.nn.sigmoid(gi[:, H:2 * H] + gh[:, H:2 * H])
    n = jnp.tanh(gi[:, 2 * H:] + r * gh[:, 2 * H:])
    hid = hid_ref[...]
    hnew = (1.0 - zg) * n + zg * hid
    hnew_ref[...] = hnew
    row = i * bn + lax.broadcasted_iota(jnp.int32, (bn, 1), 0)
    contrib = jnp.sum(jnp.where(row < n_real, hnew, 0.0), axis=0, keepdims=True)

    @pl.when(i == 0)
    def _():
        pool_ref[...] = jnp.zeros_like(pool_ref)

    pool_ref[...] += contrib


def _make_gather(npad, epad):
    rows_per_w = epad // CH // NW
    groups = rows_per_w // GRP
    mesh = plsc.VectorSubcoreMesh(core_axis_name="c", subcore_axis_name="s",
                                  num_cores=2, num_subcores=16)

    @functools.partial(
        pl.kernel,
        out_type=jax.ShapeDtypeStruct((epad, H), jnp.float32),
        mesh=mesh,
        compiler_params=pltpu.CompilerParams(use_tc_tiling_on_sc=False),
        scratch_types=[
            pltpu.VMEM((rows_per_w, CH), jnp.int32),
            pltpu.VMEM((GRP * CH, H), jnp.float32),
            pltpu.SemaphoreType.DMA,
        ],
    )
    def gather_k(table_hbm, idx_hbm, out_hbm, idx_v, rows_v, sem):
        cid = lax.axis_index("c")
        sid = lax.axis_index("s")
        wid = sid * 2 + cid
        pltpu.sync_copy(idx_hbm.at[pl.ds(wid * rows_per_w, rows_per_w)], idx_v)
        base = wid * (rows_per_w * CH)

        def body(g, carry):
            cps = [
                pltpu.async_copy(
                    table_hbm.at[idx_v.at[g * GRP + j]],
                    rows_v.at[pl.ds(j * CH, CH)], sem)
                for j in range(GRP)
            ]
            for cp in cps:
                cp.wait()
            pltpu.sync_copy(rows_v, out_hbm.at[pl.ds(base + g * (GRP * CH), GRP * CH)])
            return carry

        lax.fori_loop(0, groups, body, 0)

    return gather_k


def _make_scatter(npad, epad):
    rows_per_w = epad // CH // NW
    groups = rows_per_w // GRP
    nrows_sub = npad // 16
    mesh = plsc.VectorSubcoreMesh(core_axis_name="c", subcore_axis_name="s",
                                  num_cores=2, num_subcores=16)

    @functools.partial(
        pl.kernel,
        out_type=jax.ShapeDtypeStruct((2 * npad, H), jnp.float32),
        mesh=mesh,
        compiler_params=pltpu.CompilerParams(use_tc_tiling_on_sc=False),
        scratch_types=[
            pltpu.VMEM((rows_per_w, CH), jnp.int32),
            pltpu.VMEM((GRP * CH, H), jnp.float32),
            pltpu.VMEM_SHARED((npad, H), jnp.float32),
            pltpu.SemaphoreType.DMA,
        ],
    )
    def scatter_k(m_hbm, dst_hbm, zeros_hbm, out_hbm, idx_v, vals_v, agg_sh, sem):
        cid = lax.axis_index("c")
        sid = lax.axis_index("s")
        wid = sid * 2 + cid
        pltpu.sync_copy(zeros_hbm.at[pl.ds(sid * nrows_sub, nrows_sub)],
                        agg_sh.at[pl.ds(sid * nrows_sub, nrows_sub)])
        pltpu.sync_copy(dst_hbm.at[pl.ds(wid * rows_per_w, rows_per_w)], idx_v)
        plsc.subcore_barrier()
        base = wid * (rows_per_w * CH)

        def body(g, carry):
            pltpu.sync_copy(m_hbm.at[pl.ds(base + g * (GRP * CH), GRP * CH)], vals_v)
            cps = [
                pltpu.async_copy(
                    vals_v.at[pl.ds(j * CH, CH)],
                    agg_sh.at[idx_v.at[g * GRP + j]],
                    sem, add=True)
                for j in range(GRP)
            ]
            for cp in cps:
                cp.wait()
            return carry

        lax.fori_loop(0, groups, body, 0)
        plsc.subcore_barrier()
        pltpu.sync_copy(agg_sh.at[pl.ds(sid * nrows_sub, nrows_sub)],
                        out_hbm.at[pl.ds(cid * npad + sid * nrows_sub, nrows_sub)])

    return scatter_k


def _round_up(x, m):
    return (x + m - 1) // m * m


def kernel(node_feats, edge_feats, edge_index, Wp, bp, We1, be1, We2, be2,
           nn_bias, W_ih, W_hh, b_ih, b_hh):
    n, din = node_feats.shape
    e, ein = edge_feats.shape
    steps = 3
    npad = _round_up(n + 1, 2048)
    epad = _round_up(e, NW * CH)
    bn = 1024
    be_blk = 1024

    f32 = jnp.float32
    src = edge_index[0]
    dst = edge_index[1]
    src_pad = jnp.concatenate([src, jnp.zeros((epad - e,), jnp.int32)]).reshape(epad // CH, CH)
    dst_pad = jnp.concatenate([dst, jnp.full((epad - e,), n, jnp.int32)]).reshape(epad // CH, CH)
    ef_pad = jnp.pad(edge_feats, ((0, epad - e), (0, 0)))
    nf_pad = jnp.pad(node_feats, ((0, npad - n), (0, 0)))
    zeros_nh = jnp.zeros((npad, H), f32)

    we2p = We2.reshape(EH, H, H).transpose(0, 2, 1).reshape(EH, H * H)
    be2p = be2.reshape(H, H).T.reshape(1, H * H)
    r_mat = jnp.kron(jnp.eye(H, dtype=f32), jnp.ones((H, 1), f32))
    wih_t = W_ih.T
    whh_t = W_hh.T
    bp2 = bp[None, :]
    be1_2 = be1[None, :]
    nnb2 = nn_bias[None, :]
    bih2 = b_ih[None, :]
    bhh2 = b_hh[None, :]

    h0 = pl.pallas_call(
        _proj_body,
        grid=(npad // bn,),
        in_specs=[
            pl.BlockSpec((bn, din), lambda i: (i, 0)),
            pl.BlockSpec((din, H), lambda i: (0, 0)),
            pl.BlockSpec((1, H), lambda i: (0, 0)),
        ],
        out_specs=pl.BlockSpec((bn, H), lambda i: (i, 0)),
        out_shape=jax.ShapeDtypeStruct((npad, H), f32),
    )(nf_pad, Wp, bp2)

    gather_k = _make_gather(npad, epad)
    scatter_k = _make_scatter(npad, epad)

    msgs_call = pl.pallas_call(
        _msgs_body,
        grid=(epad // be_blk,),
        in_specs=[
            pl.BlockSpec((be_blk, ein), lambda i: (i, 0)),
            pl.BlockSpec((be_blk, H), lambda i: (i, 0)),
            pl.BlockSpec((ein, EH), lambda i: (0, 0)),
            pl.BlockSpec((1, EH), lambda i: (0, 0)),
            pl.BlockSpec((EH, H * H), lambda i: (0, 0)),
            pl.BlockSpec((1, H * H), lambda i: (0, 0)),
            pl.BlockSpec((H * H, H), lambda i: (0, 0)),
        ],
        out_specs=pl.BlockSpec((be_blk, H), lambda i: (i, 0)),
        out_shape=jax.ShapeDtypeStruct((epad, H), f32),
    )

    nblocks = npad // bn
    update_call = pl.pallas_call(
        functools.partial(_update_body, n, bn),
        grid=(nblocks,),
        in_specs=[
            pl.BlockSpec((bn, H), lambda i: (i, 0)),
            pl.BlockSpec((bn, H), lambda i: (i + nblocks, 0)),
            pl.BlockSpec((bn, H), lambda i: (i, 0)),
            pl.BlockSpec((1, H), lambda i: (0, 0)),
            pl.BlockSpec((H, 3 * H), lambda i: (0, 0)),
            pl.BlockSpec((1, 3 * H), lambda i: (0, 0)),
            pl.BlockSpec((H, 3 * H), lambda i: (0, 0)),
            pl.BlockSpec((1, 3 * H), lambda i: (0, 0)),
        ],
        out_specs=[
            pl.BlockSpec((bn, H), lambda i: (i, 0)),
            pl.BlockSpec((1, H), lambda i: (0, 0)),
        ],
        out_shape=[
            jax.ShapeDtypeStruct((npad, H), f32),
            jax.ShapeDtypeStruct((1, H), f32),
        ],
    )

    hidden = h0
    node = h0
    pooled = None
    for _ in range(steps):
        hs = gather_k(node, src_pad)
        m = msgs_call(ef_pad, hs, We1, be1_2, we2p, be2p, r_mat)
        aggp = scatter_k(m, dst_pad, zeros_nh)
        hidden, pooled = update_call(aggp, aggp, hidden, nnb2, wih_t, bih2, whh_t, bhh2)
        node = hidden
    return pooled


# speedup vs baseline: 3.7724x; 1.4972x over previous
import functools

import jax
import jax.numpy as jnp
from jax import lax
from jax.experimental import pallas as pl
from jax.experimental.pallas import tpu as pltpu
from jax.experimental.pallas import tpu_sc as plsc

H = 32
EH = 64
NW = 32
CH = 125
GRP = 10
NGRP = 4
NS = 4


def _proj_body(nf_ref, wp_ref, bp_ref, out_ref):
    out_ref[...] = jnp.maximum(
        jnp.dot(nf_ref[...], wp_ref[...], preferred_element_type=jnp.float32)
        + bp_ref[...], 0.0)


def _msgs_body(ein, rblk, ef4_ref, hsp_ref, we1_ref, be1_ref, we2p_ref,
               be2p_ref, r_ref, out_ref):
    ef_all = jnp.concatenate(
        [ef4_ref[:, q * ein:(q + 1) * ein] for q in range(NS)], axis=0)
    hs_all = jnp.concatenate(
        [hsp_ref[:, q * H:(q + 1) * H] for q in range(NS)], axis=0)
    a = jnp.maximum(
        jnp.dot(ef_all, we1_ref[...], preferred_element_type=jnp.float32)
        + be1_ref[...], 0.0).astype(jnp.bfloat16)
    c2 = (jnp.dot(a, we2p_ref[...], preferred_element_type=jnp.float32)
          + be2p_ref[...]).astype(jnp.bfloat16)
    z = c2 * jnp.tile(hs_all.astype(jnp.bfloat16), (1, H))
    m_all = jnp.dot(z, r_ref[...], preferred_element_type=jnp.float32)
    for q in range(NS):
        out_ref[:, q * H:(q + 1) * H] = m_all[q * rblk:(q + 1) * rblk]


def _update_body(agg0_ref, agg1_ref, hid_ref, nnb_ref,
                 wih_ref, bih_ref, whh_ref, bhh_ref, hnew_ref, pool_ref):
    i = pl.program_id(0)
    node = jnp.maximum(agg0_ref[...] + agg1_ref[...] + nnb_ref[...], 0.0)
    hid = hid_ref[...]
    gi = jnp.dot(node, wih_ref[...], preferred_element_type=jnp.float32) + bih_ref[...]
    gh = jnp.dot(hid, whh_ref[...], preferred_element_type=jnp.float32) + bhh_ref[...]
    r = jax.nn.sigmoid(gi[:, :H] + gh[:, :H])
    zg = jax.nn.sigmoid(gi[:, H:2 * H] + gh[:, H:2 * H])
    n = jnp.tanh(gi[:, 2 * H:] + r * gh[:, 2 * H:])
    hnew = (1.0 - zg) * n + zg * hid
    hnew_ref[...] = hnew

    @pl.when(i == 0)
    def _():
        pool_ref[...] = jnp.zeros_like(pool_ref)

    pool_ref[...] += jnp.sum(hnew, axis=0, keepdims=True)


def _make_gather(n, e):
    rows_per_w = e // CH // NW
    nrows_sub = n // 16
    gblk = GRP * CH
    es = e // NS
    wps = NW // NS
    mesh = plsc.VectorSubcoreMesh(core_axis_name="c", subcore_axis_name="s",
                                  num_cores=2, num_subcores=16)

    @functools.partial(
        pl.kernel,
        out_type=jax.ShapeDtypeStruct((es, NS * H), jnp.float32),
        mesh=mesh,
        compiler_params=pltpu.CompilerParams(use_tc_tiling_on_sc=False),
        scratch_types=[
            pltpu.VMEM((rows_per_w, CH), jnp.int32),
            pltpu.VMEM((gblk, H), jnp.float32),
            pltpu.VMEM((gblk, H), jnp.float32),
            pltpu.VMEM_SHARED((n, H), jnp.float32),
            pltpu.SemaphoreType.DMA,
            pltpu.SemaphoreType.DMA,
        ],
    )
    def gather_k(table_hbm, idx_hbm, out_hbm, idx_v, rows_a, rows_b,
                 table_sh, gsem, wsem):
        cid = lax.axis_index("c")
        sid = lax.axis_index("s")
        wid = sid * 2 + cid
        q = wid // wps
        r00 = (wid % wps) * (rows_per_w * CH)
        pltpu.sync_copy(table_hbm.at[pl.ds(sid * nrows_sub, nrows_sub)],
                        table_sh.at[pl.ds(sid * nrows_sub, nrows_sub)])
        pltpu.sync_copy(idx_hbm.at[pl.ds(wid * rows_per_w, rows_per_w)], idx_v)
        plsc.subcore_barrier()
        bufs = [rows_a, rows_b]
        wcps = []
        for g in range(NGRP):
            buf = bufs[g % 2]
            if g >= 2:
                wcps[g - 2].wait()
            cps = [
                pltpu.async_copy(
                    table_sh.at[idx_v.at[g * GRP + j]],
                    buf.at[pl.ds(j * CH, CH)], gsem)
                for j in range(GRP)
            ]
            for cp in cps:
                cp.wait()
            wcps.append(pltpu.async_copy(
                buf,
                out_hbm.at[pl.ds(r00 + g * gblk, gblk), pl.ds(q * H, H)],
                wsem))
        for g in range(max(0, NGRP - 2), NGRP):
            wcps[g].wait()

    return gather_k


def _make_scatter(n, e):
    rows_per_w = e // CH // NW
    nrows_sub = n // 16
    gblk = GRP * CH
    wps = NW // NS
    mesh = plsc.VectorSubcoreMesh(core_axis_name="c", subcore_axis_name="s",
                                  num_cores=2, num_subcores=16)

    @functools.partial(
        pl.kernel,
        out_type=[jax.ShapeDtypeStruct((n, H), jnp.float32),
                  jax.ShapeDtypeStruct((n, H), jnp.float32)],
        mesh=mesh,
        compiler_params=pltpu.CompilerParams(use_tc_tiling_on_sc=False),
        scratch_types=[
            pltpu.VMEM((rows_per_w, CH), jnp.int32),
            pltpu.VMEM((gblk, H), jnp.float32),
            pltpu.VMEM((gblk, H), jnp.float32),
            pltpu.VMEM_SHARED((n, H), jnp.float32),
            pltpu.SemaphoreType.DMA,
            pltpu.SemaphoreType.DMA,
        ],
    )
    def scatter_k(m_hbm, dst_hbm, zeros_hbm, out0_hbm, out1_hbm, idx_v,
                  vals_a, vals_b, agg_sh, vsem, ssem):
        cid = lax.axis_index("c")
        sid = lax.axis_index("s")
        wid = sid * 2 + cid
        q = wid // wps
        r00 = (wid % wps) * (rows_per_w * CH)
        pltpu.sync_copy(zeros_hbm.at[pl.ds(sid * nrows_sub, nrows_sub)],
                        agg_sh.at[pl.ds(sid * nrows_sub, nrows_sub)])
        pltpu.sync_copy(dst_hbm.at[pl.ds(wid * rows_per_w, rows_per_w)], idx_v)
        plsc.subcore_barrier()
        bufs = [vals_a, vals_b]
        vcps = {0: pltpu.async_copy(
            m_hbm.at[pl.ds(r00, gblk), pl.ds(q * H, H)], bufs[0], vsem)}
        scps = {}
        for g in range(NGRP):
            buf = bufs[g % 2]
            vcps[g].wait()
            if g + 1 < NGRP:
                if g >= 1:
                    for cp in scps[g - 1]:
                        cp.wait()
                vcps[g + 1] = pltpu.async_copy(
                    m_hbm.at[pl.ds(r00 + (g + 1) * gblk, gblk), pl.ds(q * H, H)],
                    bufs[(g + 1) % 2], vsem)
            scps[g] = [
                pltpu.async_copy(
                    buf.at[pl.ds(j * CH, CH)],
                    agg_sh.at[idx_v.at[g * GRP + j]],
                    ssem, add=True)
                for j in range(GRP)
            ]
        for g in range(max(0, NGRP - 2), NGRP):
            for cp in scps[g]:
                cp.wait()
        plsc.subcore_barrier()

        @pl.when(cid == 0)
        def _():
            pltpu.sync_copy(agg_sh.at[pl.ds(sid * nrows_sub, nrows_sub)],
                            out0_hbm.at[pl.ds(sid * nrows_sub, nrows_sub)])

        @pl.when(cid == 1)
        def _():
            pltpu.sync_copy(agg_sh.at[pl.ds(sid * nrows_sub, nrows_sub)],
                            out1_hbm.at[pl.ds(sid * nrows_sub, nrows_sub)])

    return scatter_k


def kernel(node_feats, edge_feats, edge_index, Wp, bp, We1, be1, We2, be2,
           nn_bias, W_ih, W_hh, b_ih, b_hh):
    n, din = node_feats.shape
    e, ein = edge_feats.shape
    steps = 3
    bn = 1000
    rblk = 320
    es = e // NS

    f32 = jnp.float32
    bf16 = jnp.bfloat16
    src2d = edge_index[0].reshape(e // CH, CH)
    dst2d = edge_index[1].reshape(e // CH, CH)
    zeros_nh = jnp.zeros((n, H), f32)
    ef4 = jnp.concatenate([edge_feats[qq * es:(qq + 1) * es] for qq in range(NS)],
                          axis=1).astype(bf16)

    we2p = We2.reshape(EH, H, H).transpose(0, 2, 1).reshape(EH, H * H).astype(bf16)
    be2p = be2.reshape(H, H).T.reshape(1, H * H)
    r_mat = jnp.kron(jnp.eye(H, dtype=f32), jnp.ones((H, 1), f32)).astype(bf16)
    wih_t = W_ih.T
    whh_t = W_hh.T
    bp2 = bp[None, :]
    be1_2 = be1[None, :]
    nnb2 = nn_bias[None, :]
    bih2 = b_ih[None, :]
    bhh2 = b_hh[None, :]

    h0 = pl.pallas_call(
        _proj_body,
        grid=(n // bn,),
        in_specs=[
            pl.BlockSpec((bn, din), lambda i: (i, 0)),
            pl.BlockSpec((din, H), lambda i: (0, 0)),
            pl.BlockSpec((1, H), lambda i: (0, 0)),
        ],
        out_specs=pl.BlockSpec((bn, H), lambda i: (i, 0)),
        out_shape=jax.ShapeDtypeStruct((n, H), f32),
    )(node_feats, Wp, bp2)

    we1_b = We1.astype(bf16)
    gather_k = _make_gather(n, e)
    scatter_k = _make_scatter(n, e)

    msgs_call = pl.pallas_call(
        functools.partial(_msgs_body, ein, rblk),
        grid=(es // rblk,),
        in_specs=[
            pl.BlockSpec((rblk, NS * ein), lambda i: (i, 0)),
            pl.BlockSpec((rblk, NS * H), lambda i: (i, 0)),
            pl.BlockSpec((ein, EH), lambda i: (0, 0)),
            pl.BlockSpec((1, EH), lambda i: (0, 0)),
            pl.BlockSpec((EH, H * H), lambda i: (0, 0)),
            pl.BlockSpec((1, H * H), lambda i: (0, 0)),
            pl.BlockSpec((H * H, H), lambda i: (0, 0)),
        ],
        out_specs=pl.BlockSpec((rblk, NS * H), lambda i: (i, 0)),
        out_shape=jax.ShapeDtypeStruct((es, NS * H), f32),
    )

    update_call = pl.pallas_call(
        _update_body,
        grid=(n // bn,),
        in_specs=[
            pl.BlockSpec((bn, H), lambda i: (i, 0)),
            pl.BlockSpec((bn, H), lambda i: (i, 0)),
            pl.BlockSpec((bn, H), lambda i: (i, 0)),
            pl.BlockSpec((1, H), lambda i: (0, 0)),
            pl.BlockSpec((H, 3 * H), lambda i: (0, 0)),
            pl.BlockSpec((1, 3 * H), lambda i: (0, 0)),
            pl.BlockSpec((H, 3 * H), lambda i: (0, 0)),
            pl.BlockSpec((1, 3 * H), lambda i: (0, 0)),
        ],
        out_specs=[
            pl.BlockSpec((bn, H), lambda i: (i, 0)),
            pl.BlockSpec((1, H), lambda i: (0, 0)),
        ],
        out_shape=[
            jax.ShapeDtypeStruct((n, H), f32),
            jax.ShapeDtypeStruct((1, H), f32),
        ],
    )

    hidden = h0
    node = h0
    pooled = None
    for _ in range(steps):
        hsp = gather_k(node, src2d)
        m_p = msgs_call(ef4, hsp, we1_b, be1_2, we2p, be2p, r_mat)
        agg0, agg1 = scatter_k(m_p, dst2d, zeros_nh)
        hidden, pooled = update_call(agg0, agg1, hidden, nnb2,
                                     wih_t, bih2, whh_t, bhh2)
        node = hidden
    return pooled


# speedup vs baseline: 4.1745x; 1.1066x over previous
import functools

import jax
import jax.numpy as jnp
from jax import lax
from jax.experimental import pallas as pl
from jax.experimental.pallas import tpu as pltpu
from jax.experimental.pallas import tpu_sc as plsc

H = 32
EH = 64
NW = 32
CH = 125
GRP = 10
NGRP = 4
NS = 4


def _proj_body(nf_ref, wp_ref, bp_ref, out_ref):
    out_ref[...] = jnp.maximum(
        jnp.dot(nf_ref[...], wp_ref[...], preferred_element_type=jnp.float32)
        + bp_ref[...], 0.0)


def _msgs_body(ein, rblk, ef4_ref, hsp_ref, we1_ref, be1_ref, we2p_ref,
               b2_ref, r_ref, out_ref):
    ef_all = jnp.concatenate(
        [ef4_ref[:, q * ein:(q + 1) * ein] for q in range(NS)], axis=0)
    hs_all = jnp.concatenate(
        [hsp_ref[:, q * H:(q + 1) * H] for q in range(NS)], axis=0)
    a = jnp.maximum(
        jnp.dot(ef_all, we1_ref[...], preferred_element_type=jnp.float32)
        + be1_ref[...], 0.0).astype(jnp.bfloat16)
    c2 = jnp.dot(a, we2p_ref[...],
                 preferred_element_type=jnp.float32).astype(jnp.bfloat16)
    hs_b = hs_all.astype(jnp.bfloat16)
    z = c2 * jnp.tile(hs_b, (1, H))
    m_all = (jnp.dot(z, r_ref[...], preferred_element_type=jnp.float32)
             + jnp.dot(hs_b, b2_ref[...], preferred_element_type=jnp.float32))
    for q in range(NS):
        out_ref[:, q * H:(q + 1) * H] = m_all[q * rblk:(q + 1) * rblk]


def _update_body(agg0_ref, agg1_ref, hid_ref, nnb_ref,
                 wih_ref, bih_ref, whh_ref, bhh_ref, hnew_ref, pool_ref):
    i = pl.program_id(0)
    node = jnp.maximum(agg0_ref[...] + agg1_ref[...] + nnb_ref[...], 0.0)
    hid = hid_ref[...]
    gi = jnp.dot(node, wih_ref[...], preferred_element_type=jnp.float32) + bih_ref[...]
    gh = jnp.dot(hid, whh_ref[...], preferred_element_type=jnp.float32) + bhh_ref[...]
    r = jax.nn.sigmoid(gi[:, :H] + gh[:, :H])
    zg = jax.nn.sigmoid(gi[:, H:2 * H] + gh[:, H:2 * H])
    n = jnp.tanh(gi[:, 2 * H:] + r * gh[:, 2 * H:])
    hnew = (1.0 - zg) * n + zg * hid
    hnew_ref[...] = hnew

    @pl.when(i == 0)
    def _():
        pool_ref[...] = jnp.zeros_like(pool_ref)

    pool_ref[...] += jnp.sum(hnew, axis=0, keepdims=True)


def _make_gather(n, e):
    rows_per_w = e // CH // NW
    nrows_sub = n // 16
    gblk = GRP * CH
    es = e // NS
    wps = NW // NS
    mesh = plsc.VectorSubcoreMesh(core_axis_name="c", subcore_axis_name="s",
                                  num_cores=2, num_subcores=16)

    @functools.partial(
        pl.kernel,
        out_type=jax.ShapeDtypeStruct((es, NS * H), jnp.float32),
        mesh=mesh,
        compiler_params=pltpu.CompilerParams(use_tc_tiling_on_sc=False),
        scratch_types=[
            pltpu.VMEM((rows_per_w, CH), jnp.int32),
            pltpu.VMEM((gblk, H), jnp.float32),
            pltpu.VMEM((gblk, H), jnp.float32),
            pltpu.VMEM_SHARED((n, H), jnp.float32),
            pltpu.SemaphoreType.DMA,
            pltpu.SemaphoreType.DMA,
        ],
    )
    def gather_k(table_hbm, idx_hbm, out_hbm, idx_v, rows_a, rows_b,
                 table_sh, gsem, wsem):
        cid = lax.axis_index("c")
        sid = lax.axis_index("s")
        wid = sid * 2 + cid
        q = wid // wps
        r00 = (wid % wps) * (rows_per_w * CH)
        pltpu.sync_copy(table_hbm.at[pl.ds(sid * nrows_sub, nrows_sub)],
                        table_sh.at[pl.ds(sid * nrows_sub, nrows_sub)])
        pltpu.sync_copy(idx_hbm.at[pl.ds(wid * rows_per_w, rows_per_w)], idx_v)
        plsc.subcore_barrier()
        bufs = [rows_a, rows_b]
        wcps = []
        for g in range(NGRP):
            buf = bufs[g % 2]
            if g >= 2:
                wcps[g - 2].wait()
            cps = [
                pltpu.async_copy(
                    table_sh.at[idx_v.at[g * GRP + j]],
                    buf.at[pl.ds(j * CH, CH)], gsem)
                for j in range(GRP)
            ]
            for cp in cps:
                cp.wait()
            wcps.append(pltpu.async_copy(
                buf,
                out_hbm.at[pl.ds(r00 + g * gblk, gblk), pl.ds(q * H, H)],
                wsem))
        for g in range(max(0, NGRP - 2), NGRP):
            wcps[g].wait()

    return gather_k


def _make_scatter(n, e):
    rows_per_w = e // CH // NW
    nrows_sub = n // 16
    gblk = GRP * CH
    wps = NW // NS
    mesh = plsc.VectorSubcoreMesh(core_axis_name="c", subcore_axis_name="s",
                                  num_cores=2, num_subcores=16)

    @functools.partial(
        pl.kernel,
        out_type=[jax.ShapeDtypeStruct((n, H), jnp.float32),
                  jax.ShapeDtypeStruct((n, H), jnp.float32)],
        mesh=mesh,
        compiler_params=pltpu.CompilerParams(use_tc_tiling_on_sc=False),
        scratch_types=[
            pltpu.VMEM((rows_per_w, CH), jnp.int32),
            pltpu.VMEM((gblk, H), jnp.float32),
            pltpu.VMEM((gblk, H), jnp.float32),
            pltpu.VMEM_SHARED((n, H), jnp.float32),
            pltpu.SemaphoreType.DMA,
            pltpu.SemaphoreType.DMA,
        ],
    )
    def scatter_k(m_hbm, dst_hbm, zeros_hbm, out0_hbm, out1_hbm, idx_v,
                  vals_a, vals_b, agg_sh, vsem, ssem):
        cid = lax.axis_index("c")
        sid = lax.axis_index("s")
        wid = sid * 2 + cid
        q = wid // wps
        r00 = (wid % wps) * (rows_per_w * CH)
        pltpu.sync_copy(zeros_hbm.at[pl.ds(sid * nrows_sub, nrows_sub)],
                        agg_sh.at[pl.ds(sid * nrows_sub, nrows_sub)])
        pltpu.sync_copy(dst_hbm.at[pl.ds(wid * rows_per_w, rows_per_w)], idx_v)
        plsc.subcore_barrier()
        bufs = [vals_a, vals_b]
        vcps = {0: pltpu.async_copy(
            m_hbm.at[pl.ds(r00, gblk), pl.ds(q * H, H)], bufs[0], vsem)}
        scps = {}
        for g in range(NGRP):
            buf = bufs[g % 2]
            vcps[g].wait()
            if g + 1 < NGRP:
                if g >= 1:
                    for cp in scps[g - 1]:
                        cp.wait()
                vcps[g + 1] = pltpu.async_copy(
                    m_hbm.at[pl.ds(r00 + (g + 1) * gblk, gblk), pl.ds(q * H, H)],
                    bufs[(g + 1) % 2], vsem)
            scps[g] = [
                pltpu.async_copy(
                    buf.at[pl.ds(j * CH, CH)],
                    agg_sh.at[idx_v.at[g * GRP + j]],
                    ssem, add=True)
                for j in range(GRP)
            ]
        for g in range(max(0, NGRP - 2), NGRP):
            for cp in scps[g]:
                cp.wait()
        plsc.subcore_barrier()

        @pl.when(cid == 0)
        def _():
            pltpu.sync_copy(agg_sh.at[pl.ds(sid * nrows_sub, nrows_sub)],
                            out0_hbm.at[pl.ds(sid * nrows_sub, nrows_sub)])

        @pl.when(cid == 1)
        def _():
            pltpu.sync_copy(agg_sh.at[pl.ds(sid * nrows_sub, nrows_sub)],
                            out1_hbm.at[pl.ds(sid * nrows_sub, nrows_sub)])

    return scatter_k


def kernel(node_feats, edge_feats, edge_index, Wp, bp, We1, be1, We2, be2,
           nn_bias, W_ih, W_hh, b_ih, b_hh):
    n, din = node_feats.shape
    e, ein = edge_feats.shape
    steps = 3
    bn = 1000
    rblk = 800
    es = e // NS

    f32 = jnp.float32
    bf16 = jnp.bfloat16
    src2d = edge_index[0].reshape(e // CH, CH)
    dst2d = edge_index[1].reshape(e // CH, CH)
    zeros_nh = jnp.zeros((n, H), f32)
    ef4 = jnp.concatenate([edge_feats[qq * es:(qq + 1) * es] for qq in range(NS)],
                          axis=1).astype(bf16)

    we2p = We2.reshape(EH, H, H).transpose(0, 2, 1).reshape(EH, H * H).astype(bf16)
    b2 = be2.reshape(H, H).astype(bf16)
    r_mat = jnp.kron(jnp.eye(H, dtype=f32), jnp.ones((H, 1), f32)).astype(bf16)
    wih_t = W_ih.T
    whh_t = W_hh.T
    bp2 = bp[None, :]
    be1_2 = be1[None, :]
    nnb2 = nn_bias[None, :]
    bih2 = b_ih[None, :]
    bhh2 = b_hh[None, :]

    h0 = pl.pallas_call(
        _proj_body,
        grid=(n // bn,),
        in_specs=[
            pl.BlockSpec((bn, din), lambda i: (i, 0)),
            pl.BlockSpec((din, H), lambda i: (0, 0)),
            pl.BlockSpec((1, H), lambda i: (0, 0)),
        ],
        out_specs=pl.BlockSpec((bn, H), lambda i: (i, 0)),
        out_shape=jax.ShapeDtypeStruct((n, H), f32),
    )(node_feats, Wp, bp2)

    we1_b = We1.astype(bf16)
    gather_k = _make_gather(n, e)
    scatter_k = _make_scatter(n, e)

    msgs_call = pl.pallas_call(
        functools.partial(_msgs_body, ein, rblk),
        grid=(es // rblk,),
        in_specs=[
            pl.BlockSpec((rblk, NS * ein), lambda i: (i, 0)),
            pl.BlockSpec((rblk, NS * H), lambda i: (i, 0)),
            pl.BlockSpec((ein, EH), lambda i: (0, 0)),
            pl.BlockSpec((1, EH), lambda i: (0, 0)),
            pl.BlockSpec((EH, H * H), lambda i: (0, 0)),
            pl.BlockSpec((H, H), lambda i: (0, 0)),
            pl.BlockSpec((H * H, H), lambda i: (0, 0)),
        ],
        out_specs=pl.BlockSpec((rblk, NS * H), lambda i: (i, 0)),
        out_shape=jax.ShapeDtypeStruct((es, NS * H), f32),
    )

    update_call = pl.pallas_call(
        _update_body,
        grid=(n // bn,),
        in_specs=[
            pl.BlockSpec((bn, H), lambda i: (i, 0)),
            pl.BlockSpec((bn, H), lambda i: (i, 0)),
            pl.BlockSpec((bn, H), lambda i: (i, 0)),
            pl.BlockSpec((1, H), lambda i: (0, 0)),
            pl.BlockSpec((H, 3 * H), lambda i: (0, 0)),
            pl.BlockSpec((1, 3 * H), lambda i: (0, 0)),
            pl.BlockSpec((H, 3 * H), lambda i: (0, 0)),
            pl.BlockSpec((1, 3 * H), lambda i: (0, 0)),
        ],
        out_specs=[
            pl.BlockSpec((bn, H), lambda i: (i, 0)),
            pl.BlockSpec((1, H), lambda i: (0, 0)),
        ],
        out_shape=[
            jax.ShapeDtypeStruct((n, H), f32),
            jax.ShapeDtypeStruct((1, H), f32),
        ],
    )

    hidden = h0
    node = h0
    pooled = None
    for _ in range(steps):
        hsp = gather_k(node, src2d)
        m_p = msgs_call(ef4, hsp, we1_b, be1_2, we2p, b2, r_mat)
        agg0, agg1 = scatter_k(m_p, dst2d, zeros_nh)
        hidden, pooled = update_call(agg0, agg1, hidden, nnb2,
                                     wih_t, bih2, whh_t, bhh2)
        node = hidden
    return pooled


# speedup vs baseline: 4.2884x; 1.0273x over previous
import functools

import jax
import jax.numpy as jnp
from jax import lax
from jax.experimental import pallas as pl
from jax.experimental.pallas import tpu as pltpu
from jax.experimental.pallas import tpu_sc as plsc

H = 32
EH = 64
NW = 32
CH = 125
GRP = 10
NGRP = 4
NS = 4


def _proj_body(nf_ref, wp_ref, bp_ref, out_ref):
    out_ref[...] = jnp.maximum(
        jnp.dot(nf_ref[...], wp_ref[...], preferred_element_type=jnp.float32)
        + bp_ref[...], 0.0)


def _msgs_body(ein, rblk, ef0_ref, ef1_ref, ef2_ref, ef3_ref, hsp_ref,
               we1_ref, be1_ref, we2p_ref, b2_ref, r_ref, out_ref):
    ef_all = jnp.concatenate(
        [r[...] for r in (ef0_ref, ef1_ref, ef2_ref, ef3_ref)],
        axis=0).astype(jnp.bfloat16)
    hs_all = jnp.concatenate(
        [hsp_ref[:, q * H:(q + 1) * H] for q in range(NS)], axis=0)
    a = jnp.maximum(
        jnp.dot(ef_all, we1_ref[...], preferred_element_type=jnp.float32)
        + be1_ref[...], 0.0).astype(jnp.bfloat16)
    c2 = jnp.dot(a, we2p_ref[...],
                 preferred_element_type=jnp.float32).astype(jnp.bfloat16)
    hs_b = hs_all.astype(jnp.bfloat16)
    z = c2 * jnp.tile(hs_b, (1, H))
    m_all = (jnp.dot(z, r_ref[...], preferred_element_type=jnp.float32)
             + jnp.dot(hs_b, b2_ref[...], preferred_element_type=jnp.float32))
    for q in range(NS):
        out_ref[:, q * H:(q + 1) * H] = m_all[q * rblk:(q + 1) * rblk]


def _update_body(agg0_ref, agg1_ref, hid_ref, nnb_ref,
                 wih_ref, bih_ref, whh_ref, bhh_ref, hnew_ref, pool_ref):
    i = pl.program_id(0)
    node = jnp.maximum(agg0_ref[...] + agg1_ref[...] + nnb_ref[...], 0.0)
    hid = hid_ref[...]
    gi = jnp.dot(node, wih_ref[...], preferred_element_type=jnp.float32) + bih_ref[...]
    gh = jnp.dot(hid, whh_ref[...], preferred_element_type=jnp.float32) + bhh_ref[...]
    r = jax.nn.sigmoid(gi[:, :H] + gh[:, :H])
    zg = jax.nn.sigmoid(gi[:, H:2 * H] + gh[:, H:2 * H])
    n = jnp.tanh(gi[:, 2 * H:] + r * gh[:, 2 * H:])
    hnew = (1.0 - zg) * n + zg * hid
    hnew_ref[...] = hnew

    @pl.when(i == 0)
    def _():
        pool_ref[...] = jnp.zeros_like(pool_ref)

    pool_ref[...] += jnp.sum(hnew, axis=0, keepdims=True)


def _make_gather(n, e):
    rows_per_w = e // CH // NW
    nrows_sub = n // 16
    gblk = GRP * CH
    es = e // NS
    wps = NW // NS
    mesh = plsc.VectorSubcoreMesh(core_axis_name="c", subcore_axis_name="s",
                                  num_cores=2, num_subcores=16)

    @functools.partial(
        pl.kernel,
        out_type=jax.ShapeDtypeStruct((es, NS * H), jnp.float32),
        mesh=mesh,
        compiler_params=pltpu.CompilerParams(use_tc_tiling_on_sc=False),
        scratch_types=[
            pltpu.VMEM((rows_per_w, CH), jnp.int32),
            pltpu.VMEM((gblk, H), jnp.float32),
            pltpu.VMEM((gblk, H), jnp.float32),
            pltpu.VMEM_SHARED((n, H), jnp.float32),
            pltpu.SemaphoreType.DMA,
            pltpu.SemaphoreType.DMA,
        ],
    )
    def gather_k(table_hbm, idx_hbm, out_hbm, idx_v, rows_a, rows_b,
                 table_sh, gsem, wsem):
        cid = lax.axis_index("c")
        sid = lax.axis_index("s")
        wid = sid * 2 + cid
        q = wid // wps
        r00 = (wid % wps) * (rows_per_w * CH)
        pltpu.sync_copy(table_hbm.at[pl.ds(sid * nrows_sub, nrows_sub)],
                        table_sh.at[pl.ds(sid * nrows_sub, nrows_sub)])
        pltpu.sync_copy(idx_hbm.at[pl.ds(wid * rows_per_w, rows_per_w)], idx_v)
        plsc.subcore_barrier()
        bufs = [rows_a, rows_b]
        wcps = []
        for g in range(NGRP):
            buf = bufs[g % 2]
            if g >= 2:
                wcps[g - 2].wait()
            cps = [
                pltpu.async_copy(
                    table_sh.at[idx_v.at[g * GRP + j]],
                    buf.at[pl.ds(j * CH, CH)], gsem)
                for j in range(GRP)
            ]
            for cp in cps:
                cp.wait()
            wcps.append(pltpu.async_copy(
                buf,
                out_hbm.at[pl.ds(r00 + g * gblk, gblk), pl.ds(q * H, H)],
                wsem))
        for g in range(max(0, NGRP - 2), NGRP):
            wcps[g].wait()

    return gather_k


def _make_scatter(n, e):
    rows_per_w = e // CH // NW
    nrows_sub = n // 16
    gblk = GRP * CH
    wps = NW // NS
    mesh = plsc.VectorSubcoreMesh(core_axis_name="c", subcore_axis_name="s",
                                  num_cores=2, num_subcores=16)

    @functools.partial(
        pl.kernel,
        out_type=[jax.ShapeDtypeStruct((n, H), jnp.float32),
                  jax.ShapeDtypeStruct((n, H), jnp.float32)],
        mesh=mesh,
        compiler_params=pltpu.CompilerParams(use_tc_tiling_on_sc=False),
        scratch_types=[
            pltpu.VMEM((rows_per_w, CH), jnp.int32),
            pltpu.VMEM((gblk, H), jnp.float32),
            pltpu.VMEM((gblk, H), jnp.float32),
            pltpu.VMEM_SHARED((n, H), jnp.float32),
            pltpu.SemaphoreType.DMA,
            pltpu.SemaphoreType.DMA,
        ],
    )
    def scatter_k(m_hbm, dst_hbm, zeros_hbm, out0_hbm, out1_hbm, idx_v,
                  vals_a, vals_b, agg_sh, vsem, ssem):
        cid = lax.axis_index("c")
        sid = lax.axis_index("s")
        wid = sid * 2 + cid
        q = wid // wps
        r00 = (wid % wps) * (rows_per_w * CH)
        pltpu.sync_copy(zeros_hbm.at[pl.ds(sid * nrows_sub, nrows_sub)],
                        agg_sh.at[pl.ds(sid * nrows_sub, nrows_sub)])
        pltpu.sync_copy(dst_hbm.at[pl.ds(wid * rows_per_w, rows_per_w)], idx_v)
        plsc.subcore_barrier()
        bufs = [vals_a, vals_b]
        vcps = {0: pltpu.async_copy(
            m_hbm.at[pl.ds(r00, gblk), pl.ds(q * H, H)], bufs[0], vsem)}
        scps = {}
        for g in range(NGRP):
            buf = bufs[g % 2]
            vcps[g].wait()
            if g + 1 < NGRP:
                if g >= 1:
                    for cp in scps[g - 1]:
                        cp.wait()
                vcps[g + 1] = pltpu.async_copy(
                    m_hbm.at[pl.ds(r00 + (g + 1) * gblk, gblk), pl.ds(q * H, H)],
                    bufs[(g + 1) % 2], vsem)
            scps[g] = [
                pltpu.async_copy(
                    buf.at[pl.ds(j * CH, CH)],
                    agg_sh.at[idx_v.at[g * GRP + j]],
                    ssem, add=True)
                for j in range(GRP)
            ]
        for g in range(max(0, NGRP - 2), NGRP):
            for cp in scps[g]:
                cp.wait()
        plsc.subcore_barrier()

        @pl.when(cid == 0)
        def _():
            pltpu.sync_copy(agg_sh.at[pl.ds(sid * nrows_sub, nrows_sub)],
                            out0_hbm.at[pl.ds(sid * nrows_sub, nrows_sub)])

        @pl.when(cid == 1)
        def _():
            pltpu.sync_copy(agg_sh.at[pl.ds(sid * nrows_sub, nrows_sub)],
                            out1_hbm.at[pl.ds(sid * nrows_sub, nrows_sub)])

    return scatter_k


def kernel(node_feats, edge_feats, edge_index, Wp, bp, We1, be1, We2, be2,
           nn_bias, W_ih, W_hh, b_ih, b_hh):
    n, din = node_feats.shape
    e, ein = edge_feats.shape
    steps = 3
    bn = 1000
    rblk = 800
    es = e // NS

    f32 = jnp.float32
    bf16 = jnp.bfloat16
    src2d = edge_index[0].reshape(e // CH, CH)
    dst2d = edge_index[1].reshape(e // CH, CH)
    zeros_nh = jnp.zeros((n, H), f32)

    we2p = We2.reshape(EH, H, H).transpose(0, 2, 1).reshape(EH, H * H).astype(bf16)
    b2 = be2.reshape(H, H).astype(bf16)
    r_mat = jnp.kron(jnp.eye(H, dtype=f32), jnp.ones((H, 1), f32)).astype(bf16)
    wih_t = W_ih.T
    whh_t = W_hh.T
    bp2 = bp[None, :]
    be1_2 = be1[None, :]
    nnb2 = nn_bias[None, :]
    bih2 = b_ih[None, :]
    bhh2 = b_hh[None, :]

    h0 = pl.pallas_call(
        _proj_body,
        grid=(n // bn,),
        in_specs=[
            pl.BlockSpec((bn, din), lambda i: (i, 0)),
            pl.BlockSpec((din, H), lambda i: (0, 0)),
            pl.BlockSpec((1, H), lambda i: (0, 0)),
        ],
        out_specs=pl.BlockSpec((bn, H), lambda i: (i, 0)),
        out_shape=jax.ShapeDtypeStruct((n, H), f32),
    )(node_feats, Wp, bp2)

    we1_b = We1.astype(bf16)
    gather_k = _make_gather(n, e)
    scatter_k = _make_scatter(n, e)

    msgs_call = pl.pallas_call(
        functools.partial(_msgs_body, ein, rblk),
        grid=(es // rblk,),
        in_specs=[
            pl.BlockSpec((rblk, ein), lambda i: (0 * (es // rblk) + i, 0)),
            pl.BlockSpec((rblk, ein), lambda i: (1 * (es // rblk) + i, 0)),
            pl.BlockSpec((rblk, ein), lambda i: (2 * (es // rblk) + i, 0)),
            pl.BlockSpec((rblk, ein), lambda i: (3 * (es // rblk) + i, 0)),
            pl.BlockSpec((rblk, NS * H), lambda i: (i, 0)),
            pl.BlockSpec((ein, EH), lambda i: (0, 0)),
            pl.BlockSpec((1, EH), lambda i: (0, 0)),
            pl.BlockSpec((EH, H * H), lambda i: (0, 0)),
            pl.BlockSpec((H, H), lambda i: (0, 0)),
            pl.BlockSpec((H * H, H), lambda i: (0, 0)),
        ],
        out_specs=pl.BlockSpec((rblk, NS * H), lambda i: (i, 0)),
        out_shape=jax.ShapeDtypeStruct((es, NS * H), f32),
    )

    update_call = pl.pallas_call(
        _update_body,
        grid=(n // bn,),
        in_specs=[
            pl.BlockSpec((bn, H), lambda i: (i, 0)),
            pl.BlockSpec((bn, H), lambda i: (i, 0)),
            pl.BlockSpec((bn, H), lambda i: (i, 0)),
            pl.BlockSpec((1, H), lambda i: (0, 0)),
            pl.BlockSpec((H, 3 * H), lambda i: (0, 0)),
            pl.BlockSpec((1, 3 * H), lambda i: (0, 0)),
            pl.BlockSpec((H, 3 * H), lambda i: (0, 0)),
            pl.BlockSpec((1, 3 * H), lambda i: (0, 0)),
        ],
        out_specs=[
            pl.BlockSpec((bn, H), lambda i: (i, 0)),
            pl.BlockSpec((1, H), lambda i: (0, 0)),
        ],
        out_shape=[
            jax.ShapeDtypeStruct((n, H), f32),
            jax.ShapeDtypeStruct((1, H), f32),
        ],
    )

    hidden = h0
    node = h0
    pooled = None
    for _ in range(steps):
        hsp = gather_k(node, src2d)
        m_p = msgs_call(edge_feats, edge_feats, edge_feats, edge_feats,
                        hsp, we1_b, be1_2, we2p, b2, r_mat)
        agg0, agg1 = scatter_k(m_p, dst2d, zeros_nh)
        hidden, pooled = update_call(agg0, agg1, hidden, nnb2,
                                     wih_t, bih2, whh_t, bhh2)
        node = hidden
    return pooled
